# baseline (speedup 1.0000x reference)
import jax
import jax.numpy as jnp
from jax.experimental import pallas as pl


def _out_proj_kernel(x_ref, w_ref, o_ref):
    o_ref[...] = jnp.dot(x_ref[...], w_ref[...], preferred_element_type=jnp.float32)


def kernel(node_attrs, positions, edge_index, params):
    row, col = edge_index[0], edge_index[1]
    N = node_attrs.shape[0]
    x = positions @ params["proj_w"].T
    h = node_attrs @ params["emb_in_w"].T + params["emb_in_b"]
    edge_attr = positions[row] - positions[col]
    for l in range(2):
        p = params["layers"][l]
        coord_diff = x[row] - x[col]
        radial = jnp.sum(coord_diff**2, axis=1, keepdims=True)
        ef = jnp.concatenate([h[row], h[col], radial, edge_attr], axis=1)
        ef = jax.nn.silu(ef @ p["edge_w1"].T + p["edge_b1"])
        ef = jax.nn.silu(ef @ p["edge_w2"].T + p["edge_b2"])
        cm = jax.nn.silu(ef @ p["coord_w1"].T + p["coord_b1"])
        cm = cm @ p["coord_w2"].T
        trans = coord_diff * cm
        agg_sum = jax.ops.segment_sum(trans, row, num_segments=N)
        cnt = jax.ops.segment_sum(jnp.ones_like(trans), row, num_segments=N)
        x = x + agg_sum / jnp.maximum(cnt, 1.0)
        agg = jax.ops.segment_sum(ef, row, num_segments=N)
        nf = jnp.concatenate([h, agg], axis=1)
        nf = jax.nn.silu(nf @ p["node_w1"].T + p["node_b1"])
        nf = nf @ p["node_w2"].T + p["node_b2"]
        h = h + nf
    wT = params["out_w"].T
    NB = 10000
    pred = pl.pallas_call(
        _out_proj_kernel,
        grid=(N // NB,),
        in_specs=[
            pl.BlockSpec((NB, 16), lambda i: (i, 0)),
            pl.BlockSpec((16, 3), lambda i: (0, 0)),
        ],
        out_specs=pl.BlockSpec((NB, 3), lambda i: (i, 0)),
        out_shape=jax.ShapeDtypeStruct((N, 3), jnp.float32),
    )(x, wT)
    return pred


# speedup vs baseline: 3.2305x; 3.2305x over previous
import functools

import jax
import jax.numpy as jnp
from jax import lax
from jax.experimental import pallas as pl
from jax.experimental.pallas import tpu as pltpu
from jax.experimental.pallas import tpu_sc as plsc

F32 = jnp.float32

NB = 4000
EB = 4000
CH = 128


def _silu(v):
    return v * jax.nn.sigmoid(v)


def _pre_body(pos_ref, na_ref, pw_ref, ew_ref, eb_ref, x_ref, h_ref):
    x_ref[...] = jnp.dot(pos_ref[...], pw_ref[...], preferred_element_type=F32)
    h_ref[...] = jnp.dot(na_ref[...], ew_ref[...], preferred_element_type=F32) + eb_ref[...]


def _uv_body(h_ref, pos_ref, wa_ref, wb_ref, we_ref, u_ref, v_ref):
    pe = jnp.dot(pos_ref[...], we_ref[...], preferred_element_type=F32)
    u_ref[...] = jnp.dot(h_ref[...], wa_ref[...], preferred_element_type=F32) + pe
    v_ref[...] = jnp.dot(h_ref[...], wb_ref[...], preferred_element_type=F32) - pe


def _mlp_body(gu_ref, gv_ref, gxr_ref, gxc_ref, wr_ref, b1_ref, w2_ref, b2_ref,
              wc1_ref, bc1_ref, wc2_ref, ef_ref, tr_ref, *, aug):
    cd = gxr_ref[...] - gxc_ref[...]
    rad = jnp.sum(cd * cd, axis=1, keepdims=True)
    pre1 = gu_ref[...] + gv_ref[...] + rad * wr_ref[...] + b1_ref[...]
    t = _silu(pre1)
    ef = _silu(jnp.dot(t, w2_ref[...], preferred_element_type=F32) + b2_ref[...])
    s = _silu(jnp.dot(ef, wc1_ref[...], preferred_element_type=F32) + bc1_ref[...])
    cm = jnp.sum(s * wc2_ref[...], axis=1, keepdims=True)
    tr = cd * cm
    ef_ref[...] = ef
    if aug:
        n = tr.shape[0]
        tr_ref[...] = jnp.concatenate(
            [tr, jnp.ones((n, 1), F32), jnp.zeros((n, 15), F32)], axis=1)
    else:
        tr_ref[...] = tr


def _node_body(x_ref, xagg_ref, deg_ref, h_ref, hagg_ref, wa_ref, wb_ref,
               b1_ref, w2_ref, b2_ref, xo_ref, ho_ref):
    xo_ref[...] = x_ref[...] + xagg_ref[...] / jnp.maximum(deg_ref[...], 1.0)
    nf = _silu(jnp.dot(h_ref[...], wa_ref[...], preferred_element_type=F32)
               + jnp.dot(hagg_ref[...], wb_ref[...], preferred_element_type=F32)
               + b1_ref[...])
    nf = jnp.dot(nf, w2_ref[...], preferred_element_type=F32) + b2_ref[...]
    ho_ref[...] = h_ref[...] + nf


def _out_body(x_ref, w_ref, o_ref):
    o_ref[...] = jnp.dot(x_ref[...], w_ref[...], preferred_element_type=F32)


def _node_spec(d):
    return pl.BlockSpec((NB, d), lambda i: (i, 0))


def _edge_spec(d):
    return pl.BlockSpec((EB, d), lambda i: (i, 0))


def _full_spec(r, c):
    return pl.BlockSpec((r, c), lambda i: (0, 0))


def _sc_gather_body(u_hbm, v_hbm, x_hbm, row_hbm, col_hbm,
                    gu_hbm, gv_hbm, gxr_hbm, gxc_hbm,
                    row_v, col_v, u_v, v_v, xr_v, xc_v, sem_g, sem_s,
                    *, nchunks):
    cid = lax.axis_index("c")
    sid = lax.axis_index("s")
    wid = sid * 2 + cid

    @pl.loop(wid, nchunks, step=32)
    def _chunk(c):
        base = pl.multiple_of(c * CH, CH)
        pltpu.sync_copy(row_hbm.at[pl.ds(base, CH)], row_v)
        pltpu.sync_copy(col_hbm.at[pl.ds(base, CH)], col_v)
        d1 = pltpu.async_copy(u_hbm.at[row_v], u_v, sem_g)
        d2 = pltpu.async_copy(v_hbm.at[col_v], v_v, sem_g)
        d3 = pltpu.async_copy(x_hbm.at[row_v], xr_v, sem_g)
        d4 = pltpu.async_copy(x_hbm.at[col_v], xc_v, sem_g)
        d1.wait(); d2.wait(); d3.wait(); d4.wait()
        s1 = pltpu.async_copy(u_v, gu_hbm.at[pl.ds(base, CH)], sem_s)
        s2 = pltpu.async_copy(v_v, gv_hbm.at[pl.ds(base, CH)], sem_s)
        s3 = pltpu.async_copy(xr_v, gxr_hbm.at[pl.ds(base, CH)], sem_s)
        s4 = pltpu.async_copy(xc_v, gxc_hbm.at[pl.ds(base, CH)], sem_s)
        s1.wait(); s2.wait(); s3.wait(); s4.wait()


def _sc_gather(U, V, x, row, col):
    E = row.shape[0]
    mesh = plsc.VectorSubcoreMesh(core_axis_name="c", subcore_axis_name="s")
    out_type = [
        jax.ShapeDtypeStruct((E, 32), F32),
        jax.ShapeDtypeStruct((E, 32), F32),
        jax.ShapeDtypeStruct((E, 16), F32),
        jax.ShapeDtypeStruct((E, 16), F32),
    ]
    scratch = [
        pltpu.VMEM((CH,), jnp.int32),
        pltpu.VMEM((CH,), jnp.int32),
        pltpu.VMEM((CH, 32), F32),
        pltpu.VMEM((CH, 32), F32),
        pltpu.VMEM((CH, 16), F32),
        pltpu.VMEM((CH, 16), F32),
        pltpu.SemaphoreType.DMA,
        pltpu.SemaphoreType.DMA,
    ]
    fn = pl.kernel(
        functools.partial(_sc_gather_body, nchunks=E // CH),
        out_type=out_type, mesh=mesh, scratch_types=scratch,
        compiler_params=pltpu.CompilerParams(use_tc_tiling_on_sc=False),
        name="egnn_sc_gather")
    return fn(U, V, x, row, col)


_ACC_ROWS = 51200
_ROWS_PER_TILE = _ACC_ROWS // 16
_ZC = 400


def _sc_scatter_body(row_hbm, dat_hbm, out_hbm, acc, row_v, idx_v, dat_v, tmp_v,
                     *, nchunks, n_half, width):
    cid = lax.axis_index("c")
    sid = lax.axis_index("s")
    base_n = cid * n_half

    @pl.loop(0, _ZC)
    def _zrow(i):
        for j in range(width // 16):
            tmp_v[i, pl.ds(j * 16, 16)] = jnp.zeros((16,), F32)

    tile_row0 = sid * _ROWS_PER_TILE

    @pl.loop(0, _ROWS_PER_TILE // _ZC)
    def _zchunk(k):
        off = pl.multiple_of(tile_row0 + k * _ZC, 8)
        pltpu.sync_copy(tmp_v, acc.at[pl.ds(off, _ZC)])

    plsc.subcore_barrier()

    @pl.loop(sid, nchunks, step=16)
    def _chunk(c):
        base = pl.multiple_of(c * CH, CH)
        pltpu.sync_copy(row_hbm.at[pl.ds(base, CH)], row_v)
        pltpu.sync_copy(dat_hbm.at[pl.ds(base, CH)], dat_v)
        for i in range(CH // 16):
            r = row_v[pl.ds(i * 16, 16)]
            ok = (r >= base_n) & (r < base_n + n_half)
            idx_v[pl.ds(i * 16, 16)] = jnp.where(ok, r - base_n, n_half)
        pltpu.sync_copy(dat_v, acc.at[idx_v], add=True)

    plsc.subcore_barrier()

    nreal = jnp.minimum(_ROWS_PER_TILE, jnp.maximum(0, n_half - tile_row0))

    @pl.loop(0, nreal // _ZC)
    def _wo(k):
        off = pl.multiple_of(tile_row0 + k * _ZC, 8)
        pltpu.sync_copy(acc.at[pl.ds(off, _ZC)], tmp_v)
        pltpu.sync_copy(tmp_v, out_hbm.at[pl.ds(base_n + off, _ZC)])


def _sc_scatter(row, dat, n_nodes):
    E, width = dat.shape
    n_half = n_nodes // 2
    mesh = plsc.VectorSubcoreMesh(core_axis_name="c", subcore_axis_name="s")
    scratch = [
        pltpu.VMEM_SHARED((_ACC_ROWS, width), F32),
        pltpu.VMEM((CH,), jnp.int32),
        pltpu.VMEM((CH,), jnp.int32),
        pltpu.VMEM((CH, width), F32),
        pltpu.VMEM((_ZC, width), F32),
    ]
    fn = pl.kernel(
        functools.partial(_sc_scatter_body, nchunks=E // CH,
                          n_half=n_half, width=width),
        out_type=jax.ShapeDtypeStruct((n_nodes, width), F32),
        mesh=mesh, scratch_types=scratch,
        compiler_params=pltpu.CompilerParams(use_tc_tiling_on_sc=False),
        name=f"egnn_sc_scatter{width}")
    return fn(row, dat)


def kernel(node_attrs, positions, edge_index, params):
    row, col = edge_index[0], edge_index[1]
    N = node_attrs.shape[0]
    E = row.shape[0]

    x, h = pl.pallas_call(
        _pre_body,
        grid=(N // NB,),
        in_specs=[_node_spec(3), _node_spec(3), _full_spec(3, 16),
                  _full_spec(3, 32), _full_spec(1, 32)],
        out_specs=[_node_spec(16), _node_spec(32)],
        out_shape=[jax.ShapeDtypeStruct((N, 16), F32),
                   jax.ShapeDtypeStruct((N, 32), F32)],
    )(positions, node_attrs, params["proj_w"].T, params["emb_in_w"].T,
      params["emb_in_b"][None, :])

    deg = None
    for l in range(len(params["layers"])):
        lp = params["layers"][l]
        W1 = lp["edge_w1"]
        wa, wb = W1[:, :32].T, W1[:, 32:64].T
        wr = W1[:, 64][None, :]
        we = W1[:, 65:68].T

        U, V = pl.pallas_call(
            _uv_body,
            grid=(N // NB,),
            in_specs=[_node_spec(32), _node_spec(3), _full_spec(32, 32),
                      _full_spec(32, 32), _full_spec(3, 32)],
            out_specs=[_node_spec(32), _node_spec(32)],
            out_shape=[jax.ShapeDtypeStruct((N, 32), F32),
                       jax.ShapeDtypeStruct((N, 32), F32)],
        )(h, positions, wa, wb, we)

        gu, gv, gxr, gxc = _sc_gather(U, V, x, row, col)

        aug = l == 0
        tw = 32 if aug else 16
        ef, tr = pl.pallas_call(
            functools.partial(_mlp_body, aug=aug),
            grid=(E // EB,),
            in_specs=[_edge_spec(32), _edge_spec(32), _edge_spec(16),
                      _edge_spec(16), _full_spec(1, 32), _full_spec(1, 32),
                      _full_spec(32, 32), _full_spec(1, 32), _full_spec(32, 32),
                      _full_spec(1, 32), _full_spec(1, 32)],
            out_specs=[_edge_spec(32), _edge_spec(tw)],
            out_shape=[jax.ShapeDtypeStruct((E, 32), F32),
                       jax.ShapeDtypeStruct((E, tw), F32)],
        )(gu, gv, gxr, gxc, wr, lp["edge_b1"][None, :], lp["edge_w2"].T,
          lp["edge_b2"][None, :], lp["coord_w1"].T, lp["coord_b1"][None, :],
          lp["coord_w2"])

        hagg = _sc_scatter(row, ef, N)
        ta = _sc_scatter(row, tr, N)
        if aug:
            xagg = ta[:, :16]
            deg = ta[:, 16:17]
        else:
            xagg = ta

        x, h = pl.pallas_call(
            _node_body,
            grid=(N // NB,),
            in_specs=[_node_spec(16), _node_spec(16), _node_spec(1),
                      _node_spec(32), _node_spec(32), _full_spec(32, 32),
                      _full_spec(32, 32), _full_spec(1, 32), _full_spec(32, 32),
                      _full_spec(1, 32)],
            out_specs=[_node_spec(16), _node_spec(32)],
            out_shape=[jax.ShapeDtypeStruct((N, 16), F32),
                       jax.ShapeDtypeStruct((N, 32), F32)],
        )(x, xagg, deg, h, hagg, lp["node_w1"][:, :32].T, lp["node_w1"][:, 32:].T,
          lp["node_b1"][None, :], lp["node_w2"].T, lp["node_b2"][None, :])

    pred = pl.pallas_call(
        _out_body,
        grid=(N // NB,),
        in_specs=[_node_spec(16), _full_spec(16, 3)],
        out_specs=_node_spec(3),
        out_shape=jax.ShapeDtypeStruct((N, 3), F32),
    )(x, params["out_w"].T)
    return pred


# speedup vs baseline: 5.7110x; 1.7679x over previous
import functools

import jax
import jax.numpy as jnp
from jax import lax
from jax.experimental import pallas as pl
from jax.experimental.pallas import tpu as pltpu
from jax.experimental.pallas import tpu_sc as plsc

F32 = jnp.float32

NB = 4000
EB = 8000
CH = 128
GG = 4
GS = 5


def _silu(v):
    return v * jax.nn.sigmoid(v)


def _pre_body(pos_ref, na_ref, pw_ref, ew_ref, eb_ref, x_ref, h_ref):
    x_ref[...] = jnp.dot(pos_ref[...], pw_ref[...], preferred_element_type=F32)
    h_ref[...] = jnp.dot(na_ref[...], ew_ref[...], preferred_element_type=F32) + eb_ref[...]


def _uv_body(h_ref, pos_ref, wa_ref, wb_ref, we_ref, u_ref, v_ref):
    pe = jnp.dot(pos_ref[...], we_ref[...], preferred_element_type=F32)
    u_ref[...] = jnp.dot(h_ref[...], wa_ref[...], preferred_element_type=F32) + pe
    v_ref[...] = jnp.dot(h_ref[...], wb_ref[...], preferred_element_type=F32) - pe


def _mlp_body(g_ref, wr_ref, b1_ref, w2_ref, b2_ref, wc1_ref, bc1_ref,
              wc2_ref, m_ref):
    g = g_ref[...]
    gu = g[:, 0:32]
    gv = g[:, 32:64]
    cd = g[:, 64:80] - g[:, 80:96]
    rad = jnp.sum(cd * cd, axis=1, keepdims=True)
    pre1 = gu + gv + rad * wr_ref[...] + b1_ref[...]
    t = _silu(pre1)
    ef = _silu(jnp.dot(t, w2_ref[...], preferred_element_type=F32) + b2_ref[...])
    s = _silu(jnp.dot(ef, wc1_ref[...], preferred_element_type=F32) + bc1_ref[...])
    cm = jnp.sum(s * wc2_ref[...], axis=1, keepdims=True)
    tr = cd * cm
    n = tr.shape[0]
    m_ref[...] = jnp.concatenate(
        [ef, tr, jnp.ones((n, 1), F32), jnp.zeros((n, 79), F32)], axis=1)


def _node_body(x_ref, ta_ref, h_ref, hagg_ref, wa_ref, wb_ref,
               b1_ref, w2_ref, b2_ref, xo_ref, ho_ref):
    ta = ta_ref[...]
    xagg = ta[:, 0:16]
    deg = ta[:, 16:17]
    xo_ref[...] = x_ref[...] + xagg / jnp.maximum(deg, 1.0)
    nf = _silu(jnp.dot(h_ref[...], wa_ref[...], preferred_element_type=F32)
               + jnp.dot(hagg_ref[...], wb_ref[...], preferred_element_type=F32)
               + b1_ref[...])
    nf = jnp.dot(nf, w2_ref[...], preferred_element_type=F32) + b2_ref[...]
    ho_ref[...] = h_ref[...] + nf


def _out_body(x_ref, w_ref, o_ref):
    o_ref[...] = jnp.dot(x_ref[...], w_ref[...], preferred_element_type=F32)


def _node_spec(d):
    return pl.BlockSpec((NB, d), lambda i: (i, 0))


def _edge_spec(d):
    return pl.BlockSpec((EB, d), lambda i: (i, 0))


def _full_spec(r, c):
    return pl.BlockSpec((r, c), lambda i: (0, 0))


def _sc_gather_body(u_hbm, v_hbm, x_hbm, row_hbm, col_hbm, g_hbm, *refs,
                    nchunks):
    row_v = refs[0:GG]
    col_v = refs[GG:2 * GG]
    u_v = refs[2 * GG:3 * GG]
    v_v = refs[3 * GG:4 * GG]
    xr_v = refs[4 * GG:5 * GG]
    xc_v = refs[5 * GG:6 * GG]
    sem_i = refs[6 * GG:7 * GG]
    sem_g = refs[7 * GG:8 * GG]
    sem_s = refs[8 * GG:9 * GG]

    cid = lax.axis_index("c")
    sid = lax.axis_index("s")
    wid = sid * 2 + cid
    nj = (nchunks - wid + 31) // 32
    ngroups = (nchunks + 32 * GG - 1) // (32 * GG)

    def chunk_base(g, b):
        return pl.multiple_of((wid + (g * GG + b) * 32) * CH, CH)

    def stores(g, b):
        base = chunk_base(g, b)
        return (
            pltpu.make_async_copy(u_v[b], g_hbm.at[pl.ds(base, CH), pl.ds(0, 32)], sem_s[b]),
            pltpu.make_async_copy(v_v[b], g_hbm.at[pl.ds(base, CH), pl.ds(32, 32)], sem_s[b]),
            pltpu.make_async_copy(xr_v[b], g_hbm.at[pl.ds(base, CH), pl.ds(64, 16)], sem_s[b]),
            pltpu.make_async_copy(xc_v[b], g_hbm.at[pl.ds(base, CH), pl.ds(80, 16)], sem_s[b]),
        )

    @pl.loop(0, ngroups)
    def _group(g):
        @pl.when(g > 0)
        def _():
            for b in range(GG):
                @pl.when((g - 1) * GG + b < nj)
                def _():
                    for d in stores(g - 1, b):
                        d.wait()
        for b in range(GG):
            @pl.when(g * GG + b < nj)
            def _():
                base = chunk_base(g, b)
                pltpu.async_copy(row_hbm.at[pl.ds(base, CH)], row_v[b], sem_i[b])
                pltpu.async_copy(col_hbm.at[pl.ds(base, CH)], col_v[b], sem_i[b])
        for b in range(GG):
            @pl.when(g * GG + b < nj)
            def _():
                base = chunk_base(g, b)
                pltpu.make_async_copy(row_hbm.at[pl.ds(base, CH)], row_v[b], sem_i[b]).wait()
                pltpu.make_async_copy(col_hbm.at[pl.ds(base, CH)], col_v[b], sem_i[b]).wait()
                pltpu.async_copy(u_hbm.at[row_v[b]], u_v[b], sem_g[b])
                pltpu.async_copy(v_hbm.at[col_v[b]], v_v[b], sem_g[b])
                pltpu.async_copy(x_hbm.at[row_v[b]], xr_v[b], sem_g[b])
                pltpu.async_copy(x_hbm.at[col_v[b]], xc_v[b], sem_g[b])
        for b in range(GG):
            @pl.when(g * GG + b < nj)
            def _():
                pltpu.make_async_copy(u_hbm.at[row_v[b]], u_v[b], sem_g[b]).wait()
                pltpu.make_async_copy(v_hbm.at[col_v[b]], v_v[b], sem_g[b]).wait()
                pltpu.make_async_copy(x_hbm.at[row_v[b]], xr_v[b], sem_g[b]).wait()
                pltpu.make_async_copy(x_hbm.at[col_v[b]], xc_v[b], sem_g[b]).wait()
                for d in stores(g, b):
                    d.start()

    g_last = ngroups - 1
    for b in range(GG):
        @pl.when(g_last * GG + b < nj)
        def _():
            for d in stores(g_last, b):
                d.wait()


def _sc_gather(U, V, x, row, col):
    E = row.shape[0]
    mesh = plsc.VectorSubcoreMesh(core_axis_name="c", subcore_axis_name="s")
    scratch = (
        [pltpu.VMEM((CH,), jnp.int32) for _ in range(2 * GG)]
        + [pltpu.VMEM((CH, 32), F32) for _ in range(2 * GG)]
        + [pltpu.VMEM((CH, 16), F32) for _ in range(2 * GG)]
        + [pltpu.SemaphoreType.DMA for _ in range(3 * GG)]
    )
    fn = pl.kernel(
        functools.partial(_sc_gather_body, nchunks=E // CH),
        out_type=jax.ShapeDtypeStruct((E, 128), F32),
        mesh=mesh, scratch_types=scratch,
        compiler_params=pltpu.CompilerParams(use_tc_tiling_on_sc=False),
        name="egnn_sc_gather")
    return fn(U, V, x, row, col)


_ACC_ROWS = 51200
_ROWS_PER_TILE = _ACC_ROWS // 16
_ZC = 100


def _sc_scatter_body(row_hbm, m_hbm, out_hbm, acc, *refs,
                     nchunks, n_half, col_off):
    row_v = refs[0:GS]
    idx_v = refs[GS:2 * GS]
    dat_v = refs[2 * GS:3 * GS]
    tmp_v = refs[3 * GS]
    sem_l = refs[3 * GS + 1:4 * GS + 1]
    sem_sc = refs[4 * GS + 1:5 * GS + 1]

    cid = lax.axis_index("c")
    sid = lax.axis_index("s")
    base_n = cid * n_half

    @pl.loop(0, _ZC)
    def _zrow(i):
        tmp_v[i, pl.ds(0, 16)] = jnp.zeros((16,), F32)
        tmp_v[i, pl.ds(16, 16)] = jnp.zeros((16,), F32)

    tile_row0 = sid * _ROWS_PER_TILE

    @pl.loop(0, _ROWS_PER_TILE // _ZC)
    def _zchunk(k):
        off = pl.multiple_of(tile_row0 + k * _ZC, 8)
        pltpu.sync_copy(tmp_v, acc.at[pl.ds(off, _ZC)])

    plsc.subcore_barrier()

    nj = (nchunks - sid + 15) // 16
    ngroups = (nchunks + 16 * GS - 1) // (16 * GS)

    def chunk_base(g, b):
        return pl.multiple_of((sid + (g * GS + b) * 16) * CH, CH)

    def scat_start(b):
        pltpu.async_copy(dat_v[b], acc.at[idx_v[b]], sem_sc[b], add=True)

    def scat_wait(b):
        pltpu.make_async_copy(dat_v[b], acc.at[idx_v[b]], sem_sc[b]).wait()

    @pl.loop(0, ngroups)
    def _group(g):
        @pl.when(g > 0)
        def _():
            for b in range(GS):
                @pl.when((g - 1) * GS + b < nj)
                def _():
                    scat_wait(b)
        for b in range(GS):
            @pl.when(g * GS + b < nj)
            def _():
                base = chunk_base(g, b)
                pltpu.async_copy(row_hbm.at[pl.ds(base, CH)], row_v[b], sem_l[b])
                pltpu.async_copy(
                    m_hbm.at[pl.ds(base, CH), pl.ds(col_off, 32)], dat_v[b], sem_l[b])
        for b in range(GS):
            @pl.when(g * GS + b < nj)
            def _():
                base = chunk_base(g, b)
                pltpu.make_async_copy(row_hbm.at[pl.ds(base, CH)], row_v[b], sem_l[b]).wait()
                pltpu.make_async_copy(
                    m_hbm.at[pl.ds(base, CH), pl.ds(col_off, 32)], dat_v[b], sem_l[b]).wait()
                for i in range(CH // 16):
                    r = row_v[b][pl.ds(i * 16, 16)]
                    ok = (r >= base_n) & (r < base_n + n_half)
                    idx_v[b][pl.ds(i * 16, 16)] = jnp.where(ok, r - base_n, n_half)
                scat_start(b)

    g_last = ngroups - 1
    for b in range(GS):
        @pl.when(g_last * GS + b < nj)
        def _():
            scat_wait(b)

    plsc.subcore_barrier()

    nreal = jnp.minimum(_ROWS_PER_TILE, jnp.maximum(0, n_half - tile_row0))

    @pl.loop(0, nreal // _ZC)
    def _wo(k):
        off = pl.multiple_of(tile_row0 + k * _ZC, 8)
        pltpu.sync_copy(acc.at[pl.ds(off, _ZC)], tmp_v)
        pltpu.sync_copy(tmp_v, out_hbm.at[pl.ds(base_n + off, _ZC)])


def _sc_scatter(row, m, n_nodes, col_off):
    E = row.shape[0]
    n_half = n_nodes // 2
    mesh = plsc.VectorSubcoreMesh(core_axis_name="c", subcore_axis_name="s")
    scratch = (
        [pltpu.VMEM((CH,), jnp.int32) for _ in range(2 * GS)]
        + [pltpu.VMEM((CH, 32), F32) for _ in range(GS)]
        + [pltpu.VMEM((_ZC, 32), F32)]
        + [pltpu.SemaphoreType.DMA for _ in range(2 * GS)]
    )
    fn = pl.kernel(
        functools.partial(_sc_scatter_body, nchunks=E // CH,
                          n_half=n_half, col_off=col_off),
        out_type=jax.ShapeDtypeStruct((n_nodes, 32), F32),
        mesh=mesh,
        scratch_types=[pltpu.VMEM_SHARED((_ACC_ROWS, 32), F32)] + scratch,
        compiler_params=pltpu.CompilerParams(use_tc_tiling_on_sc=False),
        name=f"egnn_sc_scatter_c{col_off}")
    return fn(row, m)


def kernel(node_attrs, positions, edge_index, params):
    row, col = edge_index[0], edge_index[1]
    N = node_attrs.shape[0]
    E = row.shape[0]

    x, h = pl.pallas_call(
        _pre_body,
        grid=(N // NB,),
        in_specs=[_node_spec(3), _node_spec(3), _full_spec(3, 16),
                  _full_spec(3, 32), _full_spec(1, 32)],
        out_specs=[_node_spec(16), _node_spec(32)],
        out_shape=[jax.ShapeDtypeStruct((N, 16), F32),
                   jax.ShapeDtypeStruct((N, 32), F32)],
    )(positions, node_attrs, params["proj_w"].T, params["emb_in_w"].T,
      params["emb_in_b"][None, :])

    for l in range(len(params["layers"])):
        lp = params["layers"][l]
        W1 = lp["edge_w1"]
        wa, wb = W1[:, :32].T, W1[:, 32:64].T
        wr = W1[:, 64][None, :]
        we = W1[:, 65:68].T

        U, V = pl.pallas_call(
            _uv_body,
            grid=(N // NB,),
            in_specs=[_node_spec(32), _node_spec(3), _full_spec(32, 32),
                      _full_spec(32, 32), _full_spec(3, 32)],
            out_specs=[_node_spec(32), _node_spec(32)],
            out_shape=[jax.ShapeDtypeStruct((N, 32), F32),
                       jax.ShapeDtypeStruct((N, 32), F32)],
        )(h, positions, wa, wb, we)

        G = _sc_gather(U, V, x, row, col)

        M = pl.pallas_call(
            _mlp_body,
            grid=(E // EB,),
            in_specs=[_edge_spec(128), _full_spec(1, 32), _full_spec(1, 32),
                      _full_spec(32, 32), _full_spec(1, 32), _full_spec(32, 32),
                      _full_spec(1, 32), _full_spec(1, 32)],
            out_specs=_edge_spec(128),
            out_shape=jax.ShapeDtypeStruct((E, 128), F32),
        )(G, wr, lp["edge_b1"][None, :], lp["edge_w2"].T,
          lp["edge_b2"][None, :], lp["coord_w1"].T, lp["coord_b1"][None, :],
          lp["coord_w2"])

        hagg = _sc_scatter(row, M, N, 0)
        ta = _sc_scatter(row, M, N, 32)

        x, h = pl.pallas_call(
            _node_body,
            grid=(N // NB,),
            in_specs=[_node_spec(16), _node_spec(32), _node_spec(32),
                      _node_spec(32), _full_spec(32, 32), _full_spec(32, 32),
                      _full_spec(1, 32), _full_spec(32, 32), _full_spec(1, 32)],
            out_specs=[_node_spec(16), _node_spec(32)],
            out_shape=[jax.ShapeDtypeStruct((N, 16), F32),
                       jax.ShapeDtypeStruct((N, 32), F32)],
        )(x, ta, h, hagg, lp["node_w1"][:, :32].T, lp["node_w1"][:, 32:].T,
          lp["node_b1"][None, :], lp["node_w2"].T, lp["node_b2"][None, :])

    pred = pl.pallas_call(
        _out_body,
        grid=(N // NB,),
        in_specs=[_node_spec(16), _full_spec(16, 3)],
        out_specs=_node_spec(3),
        out_shape=jax.ShapeDtypeStruct((N, 3), F32),
    )(x, params["out_w"].T)
    return pred
